# baseline (speedup 1.0000x reference)
import math

import numpy as np
import jax
import jax.numpy as jnp
from jax import lax
from jax.experimental import pallas as pl
from jax.experimental.pallas import tpu as pltpu
from jax.experimental.pallas import tpu_sc as plsc

_B = 8
_N_OLD = 1024
_RATIO = 2.0
_SIGMA = 0.05
_KERNEL_H = 0.1

_N_NEW_TOTAL = int(_N_OLD * _RATIO)
_N_ADDED = _N_NEW_TOTAL - _N_OLD
_TOTAL_ADDED = _B * _N_ADDED


def _threefry2x32(k1, k2, x1, x2):
    def rotl(x, d):
        return ((x << np.uint32(d)) | (x >> np.uint32(32 - d))).astype(np.uint32)

    rotations = ((13, 15, 26, 6), (17, 29, 16, 24))
    ks = [np.uint32(k1), np.uint32(k2),
          np.uint32(k1) ^ np.uint32(k2) ^ np.uint32(0x1BD11BDA)]
    with np.errstate(over="ignore"):
        x = [x1.astype(np.uint32) + ks[0], x2.astype(np.uint32) + ks[1]]

        def rounds(x, rots):
            for r in rots:
                x[0] = (x[0] + x[1]).astype(np.uint32)
                x[1] = x[0] ^ rotl(x[1], r)
            return x

        for i, rots in enumerate(
                (rotations[0], rotations[1], rotations[0],
                 rotations[1], rotations[0])):
            x = rounds(x, rots)
            x[0] = (x[0] + ks[(i + 1) % 3]).astype(np.uint32)
            x[1] = (x[1] + ks[(i + 2) % 3] + np.uint32(i + 1)).astype(np.uint32)
    return x[0], x[1]


def _iota_2x32(n):
    i = np.arange(n, dtype=np.uint64)
    return (i >> np.uint64(32)).astype(np.uint32), i.astype(np.uint32)


def _rng_split(key):
    c1, c2 = _iota_2x32(2)
    b1, b2 = _threefry2x32(key[0], key[1], c1, c2)
    return np.stack([b1, b2], axis=1)


def _random_bits32(key, n):
    c1, c2 = _iota_2x32(n)
    b1, b2 = _threefry2x32(key[0], key[1], c1, c2)
    return b1 ^ b2


def _rng_randint(key, n, minval, maxval):
    k1, k2 = _rng_split(key)
    higher = _random_bits32(k1, n)
    lower = _random_bits32(k2, n)
    span = np.uint32(maxval - minval)
    mult = np.uint32(((2 ** 16 % int(span)) ** 2) % int(span))
    with np.errstate(over="ignore"):
        off = (higher % span) * mult + (lower % span)
    return (np.int32(minval) + (off % span).astype(np.int32)).astype(np.int32)


def _erfinv_f32(x):
    x = x.astype(np.float32)
    w = -np.log((np.float32(1.0) - x) * (np.float32(1.0) + x)).astype(np.float32)
    cs_small = [2.81022636e-08, 3.43273939e-07, -3.5233877e-06,
                -4.39150654e-06, 0.00021858087, -0.00125372503,
                -0.00417768164, 0.246640727, 1.50140941]
    cs_big = [-0.000200214257, 0.000100950558, 0.00134934322,
              -0.00367342844, 0.00573950773, -0.0076224613,
              0.00943887047, 1.00167406, 2.83297682]

    def poly(cs, w):
        p = np.full_like(w, np.float32(cs[0]))
        for c in cs[1:]:
            p = np.float32(c) + p * w
        return p

    p = np.where(w < np.float32(5.0),
                 poly(cs_small, (w - np.float32(2.5)).astype(np.float32)),
                 poly(cs_big, (np.sqrt(w) - np.float32(3.0)).astype(np.float32)))
    return (p * x).astype(np.float32)


def _rng_normal_f32(key, n):
    bits = _random_bits32(key, n)
    float_bits = (bits >> np.uint32(32 - 23)) | np.uint32(0x3F800000)
    floats = float_bits.view(np.float32) - np.float32(1.0)
    lo = np.nextafter(np.float32(-1.0), np.float32(0.0), dtype=np.float32)
    hi = np.float32(1.0)
    u = np.maximum(lo, floats * (hi - lo) + lo)
    return (np.float32(math.sqrt(2.0)) * _erfinv_f32(u)).astype(np.float32)


def _make_resample_constants():
    key = np.array([0, 42], dtype=np.uint32)
    ks = _rng_split(key)
    idx = _rng_randint(ks[0], _B * _N_ADDED, 0, _N_OLD)
    noise = _rng_normal_f32(ks[1], _B * _N_ADDED) * np.float32(_SIGMA)
    return idx, noise


_IDX_CONST, _NOISE_CONST = _make_resample_constants()


_NC, _NS, _L = 2, 16, 16
_NW = _NC * _NS
_CHUNK = _TOTAL_ADDED // _NW
_SEG_W = _N_OLD // _CHUNK


def _sc_gather_body(pos_hbm, idx_hbm, out_hbm, cnt_hbm,
                    pos_v, idx_v, new_v, cnt_v):
    wid = lax.axis_index("s") * _NC + lax.axis_index("c")
    b = wid // _SEG_W
    sub = wid % _SEG_W
    src = b * _N_OLD + sub * _CHUNK
    @pl.when(wid == 0)
    def _():
        cnt_v[:] = jnp.full((_L,), _N_NEW_TOTAL, jnp.int32)
        pltpu.sync_copy(cnt_v.at[pl.ds(0, _B)], cnt_hbm)
    pltpu.sync_copy(pos_hbm.at[pl.ds(b * _N_OLD, _N_OLD)], pos_v)
    pltpu.sync_copy(idx_hbm.at[pl.ds(src, _CHUNK)], idx_v)
    for k in range(_CHUNK // _L):
        sl = pl.ds(k * _L, _L)
        new_v[sl] = plsc.load_gather(pos_v, [idx_v[sl]])
    pltpu.sync_copy(new_v, out_hbm.at[pl.ds(src, _CHUNK)])


@jax.jit
def _sc_gather(positions, idx):
    mesh = plsc.VectorSubcoreMesh(core_axis_name="c", subcore_axis_name="s")
    return pl.kernel(
        _sc_gather_body,
        out_type=(jax.ShapeDtypeStruct((_TOTAL_ADDED,), jnp.float32),
                  jax.ShapeDtypeStruct((_B,), jnp.int32)),
        mesh=mesh,
        compiler_params=pltpu.CompilerParams(needs_layout_passes=False),
        scratch_types=[
            pltpu.VMEM((_N_OLD,), jnp.float32),
            pltpu.VMEM((_CHUNK,), jnp.int32),
            pltpu.VMEM((_CHUNK,), jnp.float32),
            pltpu.VMEM((_L,), jnp.int32),
        ],
    )(positions, idx)


_C2 = -0.5 * math.log2(math.e) / (_KERNEL_H * _KERNEL_H)
_SCALE = 1.0 / (_KERNEL_H * math.sqrt(2.0 * math.pi))


_S = math.sqrt(-_C2)


def _mixture_sum(xs, ps, lw):
    n = xs.shape[0]
    a_col = lw - ps * ps
    cross = jax.lax.dot_general(
        ps + ps, xs.reshape(1, n),
        (((1,), (0,)), ((), ())),
        preferred_element_type=jnp.float32)
    arg = (cross + a_col) - (xs * xs)[None, :]
    return jnp.sum(jnp.exp2(arg), axis=0) * _SCALE


def _tc_old_body(pos_ref, w_ref, out_ref):
    p = pos_ref[:]
    ps = (p * _S).reshape(_N_OLD, 1)
    lw = jnp.log2(w_ref[:]).reshape(_N_OLD, 1)
    out_ref[:] = _mixture_sum(p * _S, ps, lw)


def _tc_new_body(pos_ref, w_ref, smp_ref, noise_ref, wold_ref,
                 out_ref, pa_ref):
    p = pos_ref[:]
    ps = (p * _S).reshape(_N_OLD, 1)
    lw = jnp.log2(w_ref[:]).reshape(_N_OLD, 1)
    xnew = smp_ref[:] + noise_ref[:]
    pa_ref[pl.ds(0, _N_OLD)] = p
    pa_ref[pl.ds(_N_OLD, _N_ADDED)] = xnew
    out_ref[pl.ds(0, _N_OLD)] = wold_ref[:]
    out_ref[pl.ds(_N_OLD, _N_ADDED)] = _mixture_sum(xnew * _S, ps, lw)


@jax.jit
def _tc_eval_old(positions, weights):
    return pl.pallas_call(
        _tc_old_body,
        grid=(_B,),
        in_specs=[
            pl.BlockSpec((_N_OLD,), lambda b: (b,)),
            pl.BlockSpec((_N_OLD,), lambda b: (b,)),
        ],
        out_specs=pl.BlockSpec((_N_OLD,), lambda b: (b,)),
        out_shape=jax.ShapeDtypeStruct((_B * _N_OLD,), jnp.float32),
    )(positions, weights)


@jax.jit
def _tc_eval_new(positions, weights, sampled, noise, w_old):
    return pl.pallas_call(
        _tc_new_body,
        grid=(_B,),
        in_specs=[
            pl.BlockSpec((_N_OLD,), lambda b: (b,)),
            pl.BlockSpec((_N_OLD,), lambda b: (b,)),
            pl.BlockSpec((_N_ADDED,), lambda b: (b,)),
            pl.BlockSpec((_N_ADDED,), lambda b: (b,)),
            pl.BlockSpec((_N_OLD,), lambda b: (b,)),
        ],
        out_specs=(pl.BlockSpec((_N_NEW_TOTAL,), lambda b: (b,)),
                   pl.BlockSpec((_N_NEW_TOTAL,), lambda b: (b,))),
        out_shape=(jax.ShapeDtypeStruct((_B * _N_NEW_TOTAL,), jnp.float32),
                   jax.ShapeDtypeStruct((_B * _N_NEW_TOTAL,), jnp.float32)),
    )(positions, weights, sampled, noise, w_old)


def kernel(positions, weights, batch_counts):
    del batch_counts
    idx = jnp.asarray(_IDX_CONST)
    noise = jnp.asarray(_NOISE_CONST)

    sampled, batch_new = _sc_gather(positions, idx)
    w_old = _tc_eval_old(positions, weights)
    w_new, pos_all = _tc_eval_new(positions, weights, sampled, noise, w_old)

    return pos_all, w_new, batch_new


# speedup vs baseline: 1.0138x; 1.0138x over previous
import math

import numpy as np
import jax
import jax.numpy as jnp
from jax import lax
from jax.experimental import pallas as pl
from jax.experimental.pallas import tpu as pltpu
from jax.experimental.pallas import tpu_sc as plsc

_B = 8
_N_OLD = 1024
_RATIO = 2.0
_SIGMA = 0.05
_KERNEL_H = 0.1

_N_NEW_TOTAL = int(_N_OLD * _RATIO)
_N_ADDED = _N_NEW_TOTAL - _N_OLD
_TOTAL_ADDED = _B * _N_ADDED


def _threefry2x32(k1, k2, x1, x2):
    def rotl(x, d):
        return ((x << np.uint32(d)) | (x >> np.uint32(32 - d))).astype(np.uint32)

    rotations = ((13, 15, 26, 6), (17, 29, 16, 24))
    ks = [np.uint32(k1), np.uint32(k2),
          np.uint32(k1) ^ np.uint32(k2) ^ np.uint32(0x1BD11BDA)]
    with np.errstate(over="ignore"):
        x = [x1.astype(np.uint32) + ks[0], x2.astype(np.uint32) + ks[1]]

        def rounds(x, rots):
            for r in rots:
                x[0] = (x[0] + x[1]).astype(np.uint32)
                x[1] = x[0] ^ rotl(x[1], r)
            return x

        for i, rots in enumerate(
                (rotations[0], rotations[1], rotations[0],
                 rotations[1], rotations[0])):
            x = rounds(x, rots)
            x[0] = (x[0] + ks[(i + 1) % 3]).astype(np.uint32)
            x[1] = (x[1] + ks[(i + 2) % 3] + np.uint32(i + 1)).astype(np.uint32)
    return x[0], x[1]


def _iota_2x32(n):
    i = np.arange(n, dtype=np.uint64)
    return (i >> np.uint64(32)).astype(np.uint32), i.astype(np.uint32)


def _rng_split(key):
    c1, c2 = _iota_2x32(2)
    b1, b2 = _threefry2x32(key[0], key[1], c1, c2)
    return np.stack([b1, b2], axis=1)


def _random_bits32(key, n):
    c1, c2 = _iota_2x32(n)
    b1, b2 = _threefry2x32(key[0], key[1], c1, c2)
    return b1 ^ b2


def _rng_randint(key, n, minval, maxval):
    k1, k2 = _rng_split(key)
    higher = _random_bits32(k1, n)
    lower = _random_bits32(k2, n)
    span = np.uint32(maxval - minval)
    mult = np.uint32(((2 ** 16 % int(span)) ** 2) % int(span))
    with np.errstate(over="ignore"):
        off = (higher % span) * mult + (lower % span)
    return (np.int32(minval) + (off % span).astype(np.int32)).astype(np.int32)


def _erfinv_f32(x):
    x = x.astype(np.float32)
    w = -np.log((np.float32(1.0) - x) * (np.float32(1.0) + x)).astype(np.float32)
    cs_small = [2.81022636e-08, 3.43273939e-07, -3.5233877e-06,
                -4.39150654e-06, 0.00021858087, -0.00125372503,
                -0.00417768164, 0.246640727, 1.50140941]
    cs_big = [-0.000200214257, 0.000100950558, 0.00134934322,
              -0.00367342844, 0.00573950773, -0.0076224613,
              0.00943887047, 1.00167406, 2.83297682]

    def poly(cs, w):
        p = np.full_like(w, np.float32(cs[0]))
        for c in cs[1:]:
            p = np.float32(c) + p * w
        return p

    p = np.where(w < np.float32(5.0),
                 poly(cs_small, (w - np.float32(2.5)).astype(np.float32)),
                 poly(cs_big, (np.sqrt(w) - np.float32(3.0)).astype(np.float32)))
    return (p * x).astype(np.float32)


def _rng_normal_f32(key, n):
    bits = _random_bits32(key, n)
    float_bits = (bits >> np.uint32(32 - 23)) | np.uint32(0x3F800000)
    floats = float_bits.view(np.float32) - np.float32(1.0)
    lo = np.nextafter(np.float32(-1.0), np.float32(0.0), dtype=np.float32)
    hi = np.float32(1.0)
    u = np.maximum(lo, floats * (hi - lo) + lo)
    return (np.float32(math.sqrt(2.0)) * _erfinv_f32(u)).astype(np.float32)


def _make_resample_constants():
    key = np.array([0, 42], dtype=np.uint32)
    ks = _rng_split(key)
    idx = _rng_randint(ks[0], _B * _N_ADDED, 0, _N_OLD)
    noise = _rng_normal_f32(ks[1], _B * _N_ADDED) * np.float32(_SIGMA)
    return idx, noise


_IDX_CONST, _NOISE_CONST = _make_resample_constants()


_NC, _NS, _L = 2, 16, 16
_NW = _NC * _NS
_CHUNK = _TOTAL_ADDED // _NW
_SEG_W = _N_OLD // _CHUNK


def _sc_gather_body(pos_hbm, idx_hbm, out_hbm, cnt_hbm,
                    pos_v, idx_v, new_v, cnt_v):
    wid = lax.axis_index("s") * _NC + lax.axis_index("c")
    b = wid // _SEG_W
    sub = wid % _SEG_W
    src = b * _N_OLD + sub * _CHUNK
    @pl.when(wid == 0)
    def _():
        cnt_v[:] = jnp.full((_L,), _N_NEW_TOTAL, jnp.int32)
        pltpu.sync_copy(cnt_v.at[pl.ds(0, _B)], cnt_hbm)
    pltpu.sync_copy(pos_hbm.at[pl.ds(b * _N_OLD, _N_OLD)], pos_v)
    pltpu.sync_copy(idx_hbm.at[pl.ds(src, _CHUNK)], idx_v)
    for k in range(_CHUNK // _L):
        sl = pl.ds(k * _L, _L)
        new_v[sl] = plsc.load_gather(pos_v, [idx_v[sl]])
    pltpu.sync_copy(new_v, out_hbm.at[pl.ds(src, _CHUNK)])


@jax.jit
def _sc_gather(positions, idx):
    mesh = plsc.VectorSubcoreMesh(core_axis_name="c", subcore_axis_name="s")
    return pl.kernel(
        _sc_gather_body,
        out_type=(jax.ShapeDtypeStruct((_TOTAL_ADDED,), jnp.float32),
                  jax.ShapeDtypeStruct((_B,), jnp.int32)),
        mesh=mesh,
        compiler_params=pltpu.CompilerParams(needs_layout_passes=False),
        scratch_types=[
            pltpu.VMEM((_N_OLD,), jnp.float32),
            pltpu.VMEM((_CHUNK,), jnp.int32),
            pltpu.VMEM((_CHUNK,), jnp.float32),
            pltpu.VMEM((_L,), jnp.int32),
        ],
    )(positions, idx)


_C2 = -0.5 * math.log2(math.e) / (_KERNEL_H * _KERNEL_H)
_SCALE = 1.0 / (_KERNEL_H * math.sqrt(2.0 * math.pi))


_S = math.sqrt(-_C2)


def _mixture_sum(xs, ps, lw):
    u = ps - xs[None, :]
    k = jnp.exp2(lw - u * u)
    return jnp.sum(k, axis=0) * _SCALE


def _tc_old_body(pos_ref, w_ref, out_ref):
    p = pos_ref[:]
    ps = (p * _S).reshape(_N_OLD, 1)
    lw = jnp.log2(w_ref[:]).reshape(_N_OLD, 1)
    out_ref[:] = _mixture_sum(p * _S, ps, lw)


def _tc_new_body(pos_ref, w_ref, smp_ref, noise_ref, wold_ref,
                 out_ref, pa_ref):
    p = pos_ref[:]
    ps = (p * _S).reshape(_N_OLD, 1)
    lw = jnp.log2(w_ref[:]).reshape(_N_OLD, 1)
    xnew = smp_ref[:] + noise_ref[:]
    pa_ref[pl.ds(0, _N_OLD)] = p
    pa_ref[pl.ds(_N_OLD, _N_ADDED)] = xnew
    out_ref[pl.ds(0, _N_OLD)] = wold_ref[:]
    out_ref[pl.ds(_N_OLD, _N_ADDED)] = _mixture_sum(xnew * _S, ps, lw)


@jax.jit
def _tc_eval_old(positions, weights):
    return pl.pallas_call(
        _tc_old_body,
        grid=(_B,),
        in_specs=[
            pl.BlockSpec((_N_OLD,), lambda b: (b,)),
            pl.BlockSpec((_N_OLD,), lambda b: (b,)),
        ],
        out_specs=pl.BlockSpec((_N_OLD,), lambda b: (b,)),
        out_shape=jax.ShapeDtypeStruct((_B * _N_OLD,), jnp.float32),
    )(positions, weights)


@jax.jit
def _tc_eval_new(positions, weights, sampled, noise, w_old):
    return pl.pallas_call(
        _tc_new_body,
        grid=(_B,),
        in_specs=[
            pl.BlockSpec((_N_OLD,), lambda b: (b,)),
            pl.BlockSpec((_N_OLD,), lambda b: (b,)),
            pl.BlockSpec((_N_ADDED,), lambda b: (b,)),
            pl.BlockSpec((_N_ADDED,), lambda b: (b,)),
            pl.BlockSpec((_N_OLD,), lambda b: (b,)),
        ],
        out_specs=(pl.BlockSpec((_N_NEW_TOTAL,), lambda b: (b,)),
                   pl.BlockSpec((_N_NEW_TOTAL,), lambda b: (b,))),
        out_shape=(jax.ShapeDtypeStruct((_B * _N_NEW_TOTAL,), jnp.float32),
                   jax.ShapeDtypeStruct((_B * _N_NEW_TOTAL,), jnp.float32)),
    )(positions, weights, sampled, noise, w_old)


def kernel(positions, weights, batch_counts):
    del batch_counts
    idx = jnp.asarray(_IDX_CONST)
    noise = jnp.asarray(_NOISE_CONST)

    sampled, batch_new = _sc_gather(positions, idx)
    w_old = _tc_eval_old(positions, weights)
    w_new, pos_all = _tc_eval_new(positions, weights, sampled, noise, w_old)

    return pos_all, w_new, batch_new


# speedup vs baseline: 1.0507x; 1.0364x over previous
import math

import numpy as np
import jax
import jax.numpy as jnp
from jax import lax
from jax.experimental import pallas as pl
from jax.experimental.pallas import tpu as pltpu
from jax.experimental.pallas import tpu_sc as plsc

_B = 8
_N_OLD = 1024
_RATIO = 2.0
_SIGMA = 0.05
_KERNEL_H = 0.1

_N_NEW_TOTAL = int(_N_OLD * _RATIO)
_N_ADDED = _N_NEW_TOTAL - _N_OLD
_TOTAL_ADDED = _B * _N_ADDED


def _threefry2x32(k1, k2, x1, x2):
    def rotl(x, d):
        return ((x << np.uint32(d)) | (x >> np.uint32(32 - d))).astype(np.uint32)

    rotations = ((13, 15, 26, 6), (17, 29, 16, 24))
    ks = [np.uint32(k1), np.uint32(k2),
          np.uint32(k1) ^ np.uint32(k2) ^ np.uint32(0x1BD11BDA)]
    with np.errstate(over="ignore"):
        x = [x1.astype(np.uint32) + ks[0], x2.astype(np.uint32) + ks[1]]

        def rounds(x, rots):
            for r in rots:
                x[0] = (x[0] + x[1]).astype(np.uint32)
                x[1] = x[0] ^ rotl(x[1], r)
            return x

        for i, rots in enumerate(
                (rotations[0], rotations[1], rotations[0],
                 rotations[1], rotations[0])):
            x = rounds(x, rots)
            x[0] = (x[0] + ks[(i + 1) % 3]).astype(np.uint32)
            x[1] = (x[1] + ks[(i + 2) % 3] + np.uint32(i + 1)).astype(np.uint32)
    return x[0], x[1]


def _iota_2x32(n):
    i = np.arange(n, dtype=np.uint64)
    return (i >> np.uint64(32)).astype(np.uint32), i.astype(np.uint32)


def _rng_split(key):
    c1, c2 = _iota_2x32(2)
    b1, b2 = _threefry2x32(key[0], key[1], c1, c2)
    return np.stack([b1, b2], axis=1)


def _random_bits32(key, n):
    c1, c2 = _iota_2x32(n)
    b1, b2 = _threefry2x32(key[0], key[1], c1, c2)
    return b1 ^ b2


def _rng_randint(key, n, minval, maxval):
    k1, k2 = _rng_split(key)
    higher = _random_bits32(k1, n)
    lower = _random_bits32(k2, n)
    span = np.uint32(maxval - minval)
    mult = np.uint32(((2 ** 16 % int(span)) ** 2) % int(span))
    with np.errstate(over="ignore"):
        off = (higher % span) * mult + (lower % span)
    return (np.int32(minval) + (off % span).astype(np.int32)).astype(np.int32)


def _erfinv_f32(x):
    x = x.astype(np.float32)
    w = -np.log((np.float32(1.0) - x) * (np.float32(1.0) + x)).astype(np.float32)
    cs_small = [2.81022636e-08, 3.43273939e-07, -3.5233877e-06,
                -4.39150654e-06, 0.00021858087, -0.00125372503,
                -0.00417768164, 0.246640727, 1.50140941]
    cs_big = [-0.000200214257, 0.000100950558, 0.00134934322,
              -0.00367342844, 0.00573950773, -0.0076224613,
              0.00943887047, 1.00167406, 2.83297682]

    def poly(cs, w):
        p = np.full_like(w, np.float32(cs[0]))
        for c in cs[1:]:
            p = np.float32(c) + p * w
        return p

    p = np.where(w < np.float32(5.0),
                 poly(cs_small, (w - np.float32(2.5)).astype(np.float32)),
                 poly(cs_big, (np.sqrt(w) - np.float32(3.0)).astype(np.float32)))
    return (p * x).astype(np.float32)


def _rng_normal_f32(key, n):
    bits = _random_bits32(key, n)
    float_bits = (bits >> np.uint32(32 - 23)) | np.uint32(0x3F800000)
    floats = float_bits.view(np.float32) - np.float32(1.0)
    lo = np.nextafter(np.float32(-1.0), np.float32(0.0), dtype=np.float32)
    hi = np.float32(1.0)
    u = np.maximum(lo, floats * (hi - lo) + lo)
    return (np.float32(math.sqrt(2.0)) * _erfinv_f32(u)).astype(np.float32)


def _make_resample_constants():
    key = np.array([0, 42], dtype=np.uint32)
    ks = _rng_split(key)
    idx = _rng_randint(ks[0], _B * _N_ADDED, 0, _N_OLD)
    noise = _rng_normal_f32(ks[1], _B * _N_ADDED) * np.float32(_SIGMA)
    kk = _rng_split(ks[0])
    k2 = (int(kk[1][0]), int(kk[1][1]))
    lo = _random_bits32(kk[1], _B * _N_ADDED) & np.uint32(1023)
    assert np.array_equal(lo.astype(np.int32), idx)
    return idx, noise, k2


_IDX_CONST, _NOISE_CONST, _TF_K2 = _make_resample_constants()


_NC, _NS, _L = 2, 16, 16
_NW = _NC * _NS
_CHUNK = _TOTAL_ADDED // _NW
_SEG_W = _N_OLD // _CHUNK


_TF_ROT = ((13, 15, 26, 6), (17, 29, 16, 24))


def _sc_threefry_idx(count):
    ks0, ks1 = np.uint32(_TF_K2[0]), np.uint32(_TF_K2[1])
    ks2 = np.uint32(ks0 ^ ks1 ^ np.uint32(0x1BD11BDA))
    x0 = jnp.full((_L,), ks0, jnp.uint32)
    x1 = count + ks1

    def rounds(x0, x1, rots):
        for r in rots:
            x0 = x0 + x1
            x1 = x0 ^ ((x1 << np.uint32(r)) | (x1 >> np.uint32(32 - r)))
        return x0, x1

    sched = ((ks1, ks2, 1), (ks2, ks0, 2), (ks0, ks1, 3),
             (ks1, ks2, 4), (ks2, ks0, 5))
    for i, (a, b, inc) in enumerate(sched):
        x0, x1 = rounds(x0, x1, _TF_ROT[i % 2])
        x0 = x0 + a
        x1 = x1 + b + np.uint32(inc)
    return ((x0 ^ x1) & np.uint32(1023)).astype(jnp.int32)


def _sc_gather_body(pos_hbm, out_hbm, cnt_hbm, pos_v, new_v, cnt_v):
    wid = lax.axis_index("s") * _NC + lax.axis_index("c")
    b = wid // _SEG_W
    sub = wid % _SEG_W
    src = b * _N_OLD + sub * _CHUNK
    @pl.when(wid == 0)
    def _():
        cnt_v[:] = jnp.full((_L,), _N_NEW_TOTAL, jnp.int32)
        pltpu.sync_copy(cnt_v.at[pl.ds(0, _B)], cnt_hbm)
    pltpu.sync_copy(pos_hbm.at[pl.ds(b * _N_OLD, _N_OLD)], pos_v)
    lane = lax.iota(jnp.uint32, _L)
    for k in range(_CHUNK // _L):
        count = lane + np.uint32(k * _L) + src.astype(jnp.uint32)
        iv = _sc_threefry_idx(count)
        new_v[pl.ds(k * _L, _L)] = plsc.load_gather(pos_v, [iv])
    pltpu.sync_copy(new_v, out_hbm.at[pl.ds(src, _CHUNK)])


@jax.jit
def _sc_gather(positions):
    mesh = plsc.VectorSubcoreMesh(core_axis_name="c", subcore_axis_name="s")
    return pl.kernel(
        _sc_gather_body,
        out_type=(jax.ShapeDtypeStruct((_TOTAL_ADDED,), jnp.float32),
                  jax.ShapeDtypeStruct((_B,), jnp.int32)),
        mesh=mesh,
        compiler_params=pltpu.CompilerParams(needs_layout_passes=False),
        scratch_types=[
            pltpu.VMEM((_N_OLD,), jnp.float32),
            pltpu.VMEM((_CHUNK,), jnp.float32),
            pltpu.VMEM((_L,), jnp.int32),
        ],
    )(positions)


_C2 = -0.5 * math.log2(math.e) / (_KERNEL_H * _KERNEL_H)
_SCALE = 1.0 / (_KERNEL_H * math.sqrt(2.0 * math.pi))


_S = math.sqrt(-_C2)


def _mixture_sum(xs, ps, lw):
    u = ps - xs[None, :]
    k = jnp.exp2(lw - u * u)
    return jnp.sum(k, axis=0) * _SCALE


def _tc_old_body(pos_ref, w_ref, out_ref):
    p = pos_ref[:]
    ps = (p * _S).reshape(_N_OLD, 1)
    lw = jnp.log2(w_ref[:]).reshape(_N_OLD, 1)
    out_ref[:] = _mixture_sum(p * _S, ps, lw)


def _tc_new_body(pos_ref, w_ref, smp_ref, noise_ref, wold_ref,
                 out_ref, pa_ref):
    p = pos_ref[:]
    ps = (p * _S).reshape(_N_OLD, 1)
    lw = jnp.log2(w_ref[:]).reshape(_N_OLD, 1)
    xnew = smp_ref[:] + noise_ref[:]
    pa_ref[pl.ds(0, _N_OLD)] = p
    pa_ref[pl.ds(_N_OLD, _N_ADDED)] = xnew
    out_ref[pl.ds(0, _N_OLD)] = wold_ref[:]
    out_ref[pl.ds(_N_OLD, _N_ADDED)] = _mixture_sum(xnew * _S, ps, lw)


@jax.jit
def _tc_eval_old(positions, weights):
    return pl.pallas_call(
        _tc_old_body,
        grid=(_B,),
        in_specs=[
            pl.BlockSpec((_N_OLD,), lambda b: (b,)),
            pl.BlockSpec((_N_OLD,), lambda b: (b,)),
        ],
        out_specs=pl.BlockSpec((_N_OLD,), lambda b: (b,)),
        out_shape=jax.ShapeDtypeStruct((_B * _N_OLD,), jnp.float32),
    )(positions, weights)


@jax.jit
def _tc_eval_new(positions, weights, sampled, noise, w_old):
    return pl.pallas_call(
        _tc_new_body,
        grid=(_B,),
        in_specs=[
            pl.BlockSpec((_N_OLD,), lambda b: (b,)),
            pl.BlockSpec((_N_OLD,), lambda b: (b,)),
            pl.BlockSpec((_N_ADDED,), lambda b: (b,)),
            pl.BlockSpec((_N_ADDED,), lambda b: (b,)),
            pl.BlockSpec((_N_OLD,), lambda b: (b,)),
        ],
        out_specs=(pl.BlockSpec((_N_NEW_TOTAL,), lambda b: (b,)),
                   pl.BlockSpec((_N_NEW_TOTAL,), lambda b: (b,))),
        out_shape=(jax.ShapeDtypeStruct((_B * _N_NEW_TOTAL,), jnp.float32),
                   jax.ShapeDtypeStruct((_B * _N_NEW_TOTAL,), jnp.float32)),
    )(positions, weights, sampled, noise, w_old)


def kernel(positions, weights, batch_counts):
    del batch_counts
    noise = jnp.asarray(_NOISE_CONST)

    sampled, batch_new = _sc_gather(positions)
    w_old = _tc_eval_old(positions, weights)
    w_new, pos_all = _tc_eval_new(positions, weights, sampled, noise, w_old)

    return pos_all, w_new, batch_new


# speedup vs baseline: 1.0621x; 1.0108x over previous
import math

import numpy as np
import jax
import jax.numpy as jnp
from jax import lax
from jax.experimental import pallas as pl
from jax.experimental.pallas import tpu as pltpu
from jax.experimental.pallas import tpu_sc as plsc

_B = 8
_N_OLD = 1024
_RATIO = 2.0
_SIGMA = 0.05
_KERNEL_H = 0.1

_N_NEW_TOTAL = int(_N_OLD * _RATIO)
_N_ADDED = _N_NEW_TOTAL - _N_OLD
_TOTAL_ADDED = _B * _N_ADDED


def _threefry2x32(k1, k2, x1, x2):
    def rotl(x, d):
        return ((x << np.uint32(d)) | (x >> np.uint32(32 - d))).astype(np.uint32)

    rotations = ((13, 15, 26, 6), (17, 29, 16, 24))
    ks = [np.uint32(k1), np.uint32(k2),
          np.uint32(k1) ^ np.uint32(k2) ^ np.uint32(0x1BD11BDA)]
    with np.errstate(over="ignore"):
        x = [x1.astype(np.uint32) + ks[0], x2.astype(np.uint32) + ks[1]]

        def rounds(x, rots):
            for r in rots:
                x[0] = (x[0] + x[1]).astype(np.uint32)
                x[1] = x[0] ^ rotl(x[1], r)
            return x

        for i, rots in enumerate(
                (rotations[0], rotations[1], rotations[0],
                 rotations[1], rotations[0])):
            x = rounds(x, rots)
            x[0] = (x[0] + ks[(i + 1) % 3]).astype(np.uint32)
            x[1] = (x[1] + ks[(i + 2) % 3] + np.uint32(i + 1)).astype(np.uint32)
    return x[0], x[1]


def _iota_2x32(n):
    i = np.arange(n, dtype=np.uint64)
    return (i >> np.uint64(32)).astype(np.uint32), i.astype(np.uint32)


def _rng_split(key):
    c1, c2 = _iota_2x32(2)
    b1, b2 = _threefry2x32(key[0], key[1], c1, c2)
    return np.stack([b1, b2], axis=1)


def _random_bits32(key, n):
    c1, c2 = _iota_2x32(n)
    b1, b2 = _threefry2x32(key[0], key[1], c1, c2)
    return b1 ^ b2


def _rng_randint(key, n, minval, maxval):
    k1, k2 = _rng_split(key)
    higher = _random_bits32(k1, n)
    lower = _random_bits32(k2, n)
    span = np.uint32(maxval - minval)
    mult = np.uint32(((2 ** 16 % int(span)) ** 2) % int(span))
    with np.errstate(over="ignore"):
        off = (higher % span) * mult + (lower % span)
    return (np.int32(minval) + (off % span).astype(np.int32)).astype(np.int32)


def _erfinv_f32(x):
    x = x.astype(np.float32)
    w = -np.log((np.float32(1.0) - x) * (np.float32(1.0) + x)).astype(np.float32)
    cs_small = [2.81022636e-08, 3.43273939e-07, -3.5233877e-06,
                -4.39150654e-06, 0.00021858087, -0.00125372503,
                -0.00417768164, 0.246640727, 1.50140941]
    cs_big = [-0.000200214257, 0.000100950558, 0.00134934322,
              -0.00367342844, 0.00573950773, -0.0076224613,
              0.00943887047, 1.00167406, 2.83297682]

    def poly(cs, w):
        p = np.full_like(w, np.float32(cs[0]))
        for c in cs[1:]:
            p = np.float32(c) + p * w
        return p

    p = np.where(w < np.float32(5.0),
                 poly(cs_small, (w - np.float32(2.5)).astype(np.float32)),
                 poly(cs_big, (np.sqrt(w) - np.float32(3.0)).astype(np.float32)))
    return (p * x).astype(np.float32)


def _rng_normal_f32(key, n):
    bits = _random_bits32(key, n)
    float_bits = (bits >> np.uint32(32 - 23)) | np.uint32(0x3F800000)
    floats = float_bits.view(np.float32) - np.float32(1.0)
    lo = np.nextafter(np.float32(-1.0), np.float32(0.0), dtype=np.float32)
    hi = np.float32(1.0)
    u = np.maximum(lo, floats * (hi - lo) + lo)
    return (np.float32(math.sqrt(2.0)) * _erfinv_f32(u)).astype(np.float32)


def _make_resample_constants():
    key = np.array([0, 42], dtype=np.uint32)
    ks = _rng_split(key)
    idx = _rng_randint(ks[0], _B * _N_ADDED, 0, _N_OLD)
    noise = _rng_normal_f32(ks[1], _B * _N_ADDED) * np.float32(_SIGMA)
    kk = _rng_split(ks[0])
    k2 = (int(kk[1][0]), int(kk[1][1]))
    lo = _random_bits32(kk[1], _B * _N_ADDED) & np.uint32(1023)
    assert np.array_equal(lo.astype(np.int32), idx)
    return idx, noise, k2


_IDX_CONST, _NOISE_CONST, _TF_K2 = _make_resample_constants()


_NC, _NS, _L = 2, 16, 16
_NW = _NC * _NS
_CHUNK = _TOTAL_ADDED // _NW
_SEG_W = _N_OLD // _CHUNK


_TF_ROT = ((13, 15, 26, 6), (17, 29, 16, 24))


def _sc_threefry_idx(count):
    ks0, ks1 = np.uint32(_TF_K2[0]), np.uint32(_TF_K2[1])
    ks2 = np.uint32(ks0 ^ ks1 ^ np.uint32(0x1BD11BDA))
    x0 = jnp.full((_L,), ks0, jnp.uint32)
    x1 = count + ks1

    def rounds(x0, x1, rots):
        for r in rots:
            x0 = x0 + x1
            x1 = x0 ^ ((x1 << np.uint32(r)) | (x1 >> np.uint32(32 - r)))
        return x0, x1

    sched = ((ks1, ks2, 1), (ks2, ks0, 2), (ks0, ks1, 3),
             (ks1, ks2, 4), (ks2, ks0, 5))
    for i, (a, b, inc) in enumerate(sched):
        x0, x1 = rounds(x0, x1, _TF_ROT[i % 2])
        x0 = x0 + a
        x1 = x1 + b + np.uint32(inc)
    return ((x0 ^ x1) & np.uint32(1023)).astype(jnp.int32)


def _sc_gather_body(pos_hbm, out_hbm, cnt_hbm, pos_v, new_v, cnt_v):
    wid = lax.axis_index("s") * _NC + lax.axis_index("c")
    b = wid // _SEG_W
    sub = wid % _SEG_W
    src = b * _N_OLD + sub * _CHUNK
    @pl.when(wid == 0)
    def _():
        cnt_v[:] = jnp.full((_L,), _N_NEW_TOTAL, jnp.int32)
        pltpu.sync_copy(cnt_v.at[pl.ds(0, _B)], cnt_hbm)
    pltpu.sync_copy(pos_hbm.at[pl.ds(b * _N_OLD, _N_OLD)], pos_v)
    lane = lax.iota(jnp.uint32, _L)
    for k in range(_CHUNK // _L):
        count = lane + np.uint32(k * _L) + src.astype(jnp.uint32)
        iv = _sc_threefry_idx(count)
        new_v[pl.ds(k * _L, _L)] = plsc.load_gather(pos_v, [iv])
    pltpu.sync_copy(new_v, out_hbm.at[pl.ds(src, _CHUNK)])


@jax.jit
def _sc_gather(positions):
    mesh = plsc.VectorSubcoreMesh(core_axis_name="c", subcore_axis_name="s")
    return pl.kernel(
        _sc_gather_body,
        out_type=(jax.ShapeDtypeStruct((_TOTAL_ADDED,), jnp.float32),
                  jax.ShapeDtypeStruct((_B,), jnp.int32)),
        mesh=mesh,
        compiler_params=pltpu.CompilerParams(needs_layout_passes=False),
        scratch_types=[
            pltpu.VMEM((_N_OLD,), jnp.float32),
            pltpu.VMEM((_CHUNK,), jnp.float32),
            pltpu.VMEM((_L,), jnp.int32),
        ],
    )(positions)


_C2 = -0.5 * math.log2(math.e) / (_KERNEL_H * _KERNEL_H)
_SCALE = 1.0 / (_KERNEL_H * math.sqrt(2.0 * math.pi))


_S = math.sqrt(-_C2)


def _mixture_sum(xs, ps, lw):
    u = ps - xs[None, :]
    k = jnp.exp2(lw - u * u)
    return jnp.sum(k, axis=0) * _SCALE


_SPS = 2


def _tc_old_body(pos_ref, w_ref, out_ref):
    for s in range(_SPS):
        sl = pl.ds(s * _N_OLD, _N_OLD)
        p = pos_ref[sl]
        ps = (p * _S).reshape(_N_OLD, 1)
        lw = jnp.log2(w_ref[sl]).reshape(_N_OLD, 1)
        out_ref[sl] = _mixture_sum(p * _S, ps, lw)


def _tc_new_body(pos_ref, w_ref, smp_ref, noise_ref, wold_ref,
                 out_ref, pa_ref):
    for s in range(_SPS):
        slo = pl.ds(s * _N_OLD, _N_OLD)
        sla = pl.ds(s * _N_ADDED, _N_ADDED)
        p = pos_ref[slo]
        ps = (p * _S).reshape(_N_OLD, 1)
        lw = jnp.log2(w_ref[slo]).reshape(_N_OLD, 1)
        xnew = smp_ref[sla] + noise_ref[sla]
        base = s * _N_NEW_TOTAL
        pa_ref[pl.ds(base, _N_OLD)] = p
        pa_ref[pl.ds(base + _N_OLD, _N_ADDED)] = xnew
        out_ref[pl.ds(base, _N_OLD)] = wold_ref[slo]
        out_ref[pl.ds(base + _N_OLD, _N_ADDED)] = _mixture_sum(xnew * _S, ps, lw)


@jax.jit
def _tc_eval_old(positions, weights):
    return pl.pallas_call(
        _tc_old_body,
        grid=(_B // _SPS,),
        in_specs=[
            pl.BlockSpec((_SPS * _N_OLD,), lambda b: (b,)),
            pl.BlockSpec((_SPS * _N_OLD,), lambda b: (b,)),
        ],
        out_specs=pl.BlockSpec((_SPS * _N_OLD,), lambda b: (b,)),
        out_shape=jax.ShapeDtypeStruct((_B * _N_OLD,), jnp.float32),
    )(positions, weights)


@jax.jit
def _tc_eval_new(positions, weights, sampled, noise, w_old):
    return pl.pallas_call(
        _tc_new_body,
        grid=(_B // _SPS,),
        in_specs=[
            pl.BlockSpec((_SPS * _N_OLD,), lambda b: (b,)),
            pl.BlockSpec((_SPS * _N_OLD,), lambda b: (b,)),
            pl.BlockSpec((_SPS * _N_ADDED,), lambda b: (b,)),
            pl.BlockSpec((_SPS * _N_ADDED,), lambda b: (b,)),
            pl.BlockSpec((_SPS * _N_OLD,), lambda b: (b,)),
        ],
        out_specs=(pl.BlockSpec((_SPS * _N_NEW_TOTAL,), lambda b: (b,)),
                   pl.BlockSpec((_SPS * _N_NEW_TOTAL,), lambda b: (b,))),
        out_shape=(jax.ShapeDtypeStruct((_B * _N_NEW_TOTAL,), jnp.float32),
                   jax.ShapeDtypeStruct((_B * _N_NEW_TOTAL,), jnp.float32)),
    )(positions, weights, sampled, noise, w_old)


def kernel(positions, weights, batch_counts):
    del batch_counts
    noise = jnp.asarray(_NOISE_CONST)

    sampled, batch_new = _sc_gather(positions)
    w_old = _tc_eval_old(positions, weights)
    w_new, pos_all = _tc_eval_new(positions, weights, sampled, noise, w_old)

    return pos_all, w_new, batch_new
